# baseline (speedup 1.0000x reference)
import jax
import jax.numpy as jnp
from jax import lax
from jax.experimental import pallas as pl
from jax.experimental.pallas import tpu as pltpu
from jax.experimental.pallas import tpu_sc as plsc

D = 32
B = 16384
NW = 32
BPW = B // NW
NCHUNK = 4
CHUNK = BPW // NCHUNK
L = 16


def _gmf_body(ui_hbm, ii_hbm, ut_hbm, it_hbm, w_hbm, b_hbm, out_hbm,
              idx_u, idx_i, rows_u, rows_i, outv, wv, bv,
              sem_u, sem_i):
    wid = lax.axis_index("s") * 2 + lax.axis_index("c")
    base = wid * BPW

    pltpu.sync_copy(ui_hbm.at[wid], idx_u)
    pltpu.sync_copy(ii_hbm.at[wid], idx_i)
    pltpu.sync_copy(w_hbm, wv)
    pltpu.sync_copy(b_hbm, bv)

    copies = []
    for j in range(NCHUNK):
        copies.append(pltpu.async_copy(
            ut_hbm.at[idx_u.at[j]], rows_u.at[pl.ds(j * CHUNK, CHUNK)], sem_u))
        copies.append(pltpu.async_copy(
            it_hbm.at[idx_i.at[j]], rows_i.at[pl.ds(j * CHUNK, CHUNK)], sem_i))
    for c in copies:
        c.wait()

    w_lo = wv[pl.ds(0, L)]
    w_hi = wv[pl.ds(L, L)]
    b_v = bv[...]

    lanes = lax.iota(jnp.int32, L)

    def grp_body(g, carry):
        v = jnp.zeros((L,), jnp.float32)
        for l in range(L):
            r = g * L + l
            p = (rows_u[r, pl.ds(0, L)] * rows_i[r, pl.ds(0, L)] * w_lo
                 + rows_u[r, pl.ds(L, L)] * rows_i[r, pl.ds(L, L)] * w_hi)
            v = jnp.where(lanes == l, jnp.sum(p), v)
        logits = v + b_v
        outv[pl.ds(g * L, L)] = 1.0 / (1.0 + jnp.exp(-logits))
        return carry

    lax.fori_loop(0, BPW // L, grp_body, 0)

    pltpu.sync_copy(outv, out_hbm.at[pl.ds(base, BPW)])


@jax.jit
def _gmf(ui3, ii3, user_table, item_table, w32, b16):
    mesh = plsc.VectorSubcoreMesh(core_axis_name="c", subcore_axis_name="s")
    f = pl.kernel(
        _gmf_body,
        out_type=jax.ShapeDtypeStruct((B,), jnp.float32),
        mesh=mesh,
        compiler_params=pltpu.CompilerParams(
            use_tc_tiling_on_sc=False, needs_layout_passes=False),
        scratch_types=[
            pltpu.VMEM((NCHUNK, CHUNK), jnp.int32),
            pltpu.VMEM((NCHUNK, CHUNK), jnp.int32),
            pltpu.VMEM((BPW, D), jnp.float32),
            pltpu.VMEM((BPW, D), jnp.float32),
            pltpu.VMEM((BPW,), jnp.float32),
            pltpu.VMEM((D,), jnp.float32),
            pltpu.VMEM((L,), jnp.float32),
            pltpu.SemaphoreType.DMA,
            pltpu.SemaphoreType.DMA,
        ],
    )
    return f(ui3, ii3, user_table, item_table, w32, b16)


def kernel(user_indices, item_indices, user_table, item_table, W, b):
    ui3 = user_indices.astype(jnp.int32).reshape(NW, NCHUNK, CHUNK)
    ii3 = item_indices.astype(jnp.int32).reshape(NW, NCHUNK, CHUNK)
    w32 = W.reshape(D)
    b16 = jnp.broadcast_to(b, (L,))
    out = _gmf(ui3, ii3, user_table, item_table, w32, b16)
    return out.reshape(B, 1)
